# baseline (speedup 1.0000x reference)
import functools

import jax
import jax.numpy as jnp
from jax import lax
from jax.experimental import pallas as pl
from jax.experimental.pallas import tpu as pltpu
from jax.experimental.pallas import tpu_sc as plsc

VOCAB = 100000
EMBED = 128
PROJ = 512
LN_EPS = 1e-5

NC = 2
NS = 16
NW = NC * NS

LPAD = 56
CHUNK = 128
BB = 16


def _make_sc_gather(n_rows, n_chunks):
    per_w = n_chunks * CHUNK
    mesh = plsc.VectorSubcoreMesh(core_axis_name="c", subcore_axis_name="s")

    @functools.partial(
        pl.kernel,
        out_type=jax.ShapeDtypeStruct((n_rows, EMBED), jnp.float32),
        mesh=mesh,
        scratch_types=[
            pltpu.VMEM((n_chunks, CHUNK), jnp.int32),
            pltpu.VMEM((CHUNK, EMBED), jnp.float32),
            pltpu.SemaphoreType.DMA,
        ],
    )
    def gather_kernel(table_hbm, idx_hbm, out_hbm, idx_v, rows_v, sem):
        wid = lax.axis_index("s") * NC + lax.axis_index("c")
        pltpu.sync_copy(idx_hbm.at[wid], idx_v)
        base = wid * per_w

        def step(j, carry):
            pltpu.async_copy(table_hbm.at[idx_v.at[j]], rows_v, sem).wait()
            pltpu.sync_copy(rows_v, out_hbm.at[pl.ds(base + j * CHUNK, CHUNK)])
            return carry

        lax.fori_loop(0, n_chunks, step, 0)

    return gather_kernel


def _proj_ln_body(g_ref, w_ref, b_ref, gamma_ref, beta_ref, out_ref):
    x = g_ref[...]
    h = lax.dot_general(
        x, w_ref[...],
        dimension_numbers=(((1,), (1,)), ((), ())),
        preferred_element_type=jnp.float32,
    )
    h = h + b_ref[...]
    mu = jnp.mean(h, axis=-1, keepdims=True)
    var = jnp.mean((h - mu) ** 2, axis=-1, keepdims=True)
    y = (h - mu) * lax.rsqrt(var + LN_EPS) * gamma_ref[...] + beta_ref[...]
    y3 = y.reshape(BB, LPAD, PROJ)
    out_ref[...] = y3[:, :50, :]


def _project_ln(g, W, b, gamma, beta, B, L):
    return pl.pallas_call(
        _proj_ln_body,
        grid=(B // BB,),
        in_specs=[
            pl.BlockSpec((BB * LPAD, EMBED), lambda i: (i, 0)),
            pl.BlockSpec((PROJ, EMBED), lambda i: (0, 0)),
            pl.BlockSpec((1, PROJ), lambda i: (0, 0)),
            pl.BlockSpec((1, PROJ), lambda i: (0, 0)),
            pl.BlockSpec((1, PROJ), lambda i: (0, 0)),
        ],
        out_specs=pl.BlockSpec((BB, L, PROJ), lambda i: (i, 0, 0)),
        out_shape=jax.ShapeDtypeStruct((B, L, PROJ), jnp.float32),
    )(g, W, b.reshape(1, PROJ), gamma.reshape(1, PROJ), beta.reshape(1, PROJ))


NSPLIT = 2


def kernel(texts, table, W, b, gamma, beta):
    B, L = texts.shape
    pad_idx = (jnp.arange(B * (LPAD - L), dtype=jnp.int32) % VOCAB).reshape(
        B, LPAD - L)
    texts_p = jnp.concatenate([texts.astype(jnp.int32), pad_idx], axis=1)
    bs = B // NSPLIT
    n_rows = bs * LPAD
    n_chunks = n_rows // (NW * CHUNK)
    parts = []
    for k in range(NSPLIT):
        idx = texts_p[k * bs:(k + 1) * bs].reshape(NW, n_chunks, CHUNK)
        g = _make_sc_gather(n_rows, n_chunks)(table, idx)
        parts.append(_project_ln(g, W, b, gamma, beta, bs, L))
    return lax.concatenate(parts, 0)


# speedup vs baseline: 1.3086x; 1.3086x over previous
import functools

import jax
import jax.numpy as jnp
from jax import lax
from jax.experimental import pallas as pl
from jax.experimental.pallas import tpu as pltpu
from jax.experimental.pallas import tpu_sc as plsc

VOCAB = 100000
EMBED = 128
PROJ = 512
LN_EPS = 1e-5

NC = 2
NS = 16
NW = NC * NS

LPAD = 56
CHUNK = 128
ROWS_F = 2000


def _project_ln_body(table_ref, w_ref, b_ref, gamma_ref, beta_ref, out_ref):
    h = lax.dot_general(
        table_ref[...], w_ref[...],
        dimension_numbers=(((1,), (1,)), ((), ())),
        preferred_element_type=jnp.float32,
    )
    h = h + b_ref[...]
    mu = jnp.mean(h, axis=-1, keepdims=True)
    var = jnp.mean((h - mu) ** 2, axis=-1, keepdims=True)
    out_ref[...] = (h - mu) * lax.rsqrt(var + LN_EPS) * gamma_ref[...] + beta_ref[...]


def _project_ln(table, W, b, gamma, beta):
    return pl.pallas_call(
        _project_ln_body,
        grid=(VOCAB // ROWS_F,),
        in_specs=[
            pl.BlockSpec((ROWS_F, EMBED), lambda i: (i, 0)),
            pl.BlockSpec((PROJ, EMBED), lambda i: (0, 0)),
            pl.BlockSpec((1, PROJ), lambda i: (0, 0)),
            pl.BlockSpec((1, PROJ), lambda i: (0, 0)),
            pl.BlockSpec((1, PROJ), lambda i: (0, 0)),
        ],
        out_specs=pl.BlockSpec((ROWS_F, PROJ), lambda i: (i, 0)),
        out_shape=jax.ShapeDtypeStruct((VOCAB, PROJ), jnp.float32),
    )(table, W, b.reshape(1, PROJ), gamma.reshape(1, PROJ), beta.reshape(1, PROJ))


def _make_sc_gather(n_rows, n_chunks):
    per_w = n_chunks * CHUNK
    mesh = plsc.VectorSubcoreMesh(core_axis_name="c", subcore_axis_name="s")

    @functools.partial(
        pl.kernel,
        out_type=jax.ShapeDtypeStruct((n_rows, PROJ), jnp.float32),
        mesh=mesh,
        scratch_types=[
            pltpu.VMEM((n_chunks, CHUNK), jnp.int32),
            pltpu.VMEM((CHUNK, PROJ), jnp.float32),
            pltpu.SemaphoreType.DMA,
        ],
    )
    def gather_kernel(f_hbm, idx_hbm, out_hbm, idx_v, rows_v, sem):
        wid = lax.axis_index("s") * NC + lax.axis_index("c")
        pltpu.sync_copy(idx_hbm.at[wid], idx_v)
        base = wid * per_w

        def step(j, carry):
            pltpu.async_copy(f_hbm.at[idx_v.at[j]], rows_v, sem).wait()
            pltpu.sync_copy(rows_v, out_hbm.at[pl.ds(base + j * CHUNK, CHUNK)])
            return carry

        lax.fori_loop(0, n_chunks, step, 0)

    return gather_kernel


def kernel(texts, table, W, b, gamma, beta):
    B, L = texts.shape
    f = _project_ln(table, W, b, gamma, beta)
    pad_idx = (jnp.arange(B * (LPAD - L), dtype=jnp.int32) % VOCAB).reshape(
        B, LPAD - L)
    texts_p = jnp.concatenate([texts.astype(jnp.int32), pad_idx], axis=1)
    n_rows = B * LPAD
    n_chunks = n_rows // (NW * CHUNK)
    idx = texts_p.reshape(NW, n_chunks, CHUNK)
    gp = _make_sc_gather(n_rows, n_chunks)(f, idx)
    return gp.reshape(B, LPAD, PROJ)[:, :L, :]
